# baseline (speedup 1.0000x reference)
import functools

import jax
import jax.numpy as jnp
from jax import lax
from jax.experimental import pallas as pl
from jax.experimental.pallas import tpu as pltpu, tpu_sc as plsc

D = 1024
DW = D // 2

_info = plsc.get_sparse_core_info()
NC, NS, L = _info.num_cores, _info.num_subcores, _info.num_lanes
NW = NC * NS

B = 4 * 2048
B_PER_W = B // NW
CHUNK = 32
N_CHUNKS = B_PER_W // CHUNK
ITEMS_PER_HALF = 16 * 8

_mesh = plsc.VectorSubcoreMesh(core_axis_name="c", subcore_axis_name="s")


@functools.partial(
    pl.kernel,
    mesh=_mesh,
    out_type=jax.ShapeDtypeStruct((B * DW // 1024, 8, 128), jnp.int32),
    scratch_types=[
        pltpu.VMEM((N_CHUNKS, CHUNK), jnp.int32),
        pltpu.VMEM((2 * N_CHUNKS, 128), jnp.int32),
        pltpu.VMEM((2 * CHUNK * 4, 128), jnp.float32),
        pltpu.VMEM((2 * CHUNK * 4, 128), jnp.float32),
        pltpu.VMEM((16, 8, 128), jnp.int32),
        pltpu.VMEM((16, 8, 128), jnp.int32),
        pltpu.SemaphoreType.DMA,
        pltpu.SemaphoreType.DMA,
    ],
    compiler_params=pltpu.CompilerParams(
        use_tc_tiling_on_sc=False, needs_layout_passes=False),
)
def _embed_sc(idx_hbm, table_hbm, out_hbm, idx_v, item_v, rows0, rows1,
              outb0, outb1, gsem, osem):
    wid = lax.axis_index("s") * NC + lax.axis_index("c")
    pltpu.sync_copy(idx_hbm.at[wid], idx_v)

    iota = lax.iota(jnp.int32, L)
    ev_lane = iota * 2

    for g in range(N_CHUNKS):
        for h in range(2):
            v = idx_v[g, pl.ds(16 * h, 16)]
            base = ((v >> 3) << 6) + (v & 7)
            for c in range(8):
                item_v[2 * g + h, pl.ds(16 * c, 16)] = base + 8 * c

    rows_bufs = (rows0, rows1)
    out_bufs = (outb0, outb1)

    def start_gather(g):
        buf = rows_bufs[g % 2]
        return (
            pltpu.async_copy(table_hbm.at[item_v.at[2 * g]],
                             buf.at[pl.ds(0, 128)], gsem),
            pltpu.async_copy(table_hbm.at[item_v.at[2 * g + 1]],
                             buf.at[pl.ds(128, 128)], gsem),
        )

    def convert_chunk(rows_ref, out_ref):
        def row_body(r, _):
            d0_base = ((r >> 4) << 7) + (r & 15)
            a_out = r >> 1
            b_par = (r & 1) * 4
            for j in range(32):
                d0 = lax.broadcast(d0_base + (j // 4) * 16, (L,))
                col = 32 * (j % 4)
                a = plsc.load_gather(rows_ref, [d0, col + ev_lane])
                b = plsc.load_gather(rows_ref, [d0, col + ev_lane + 1])
                ua = plsc.bitcast(a, jnp.int32)
                ub = plsc.bitcast(b, jnp.int32)
                ta = ua + 0x7FFF + ((ua >> 16) & 1)
                tb = ub + 0x7FFF + ((ub >> 16) & 1)
                word = (lax.shift_right_logical(ta, 16)
                        | (tb & jnp.int32(-0x10000)))
                out_ref[a_out, b_par + j // 8, pl.ds(16 * (j % 8), L)] = word
            return 0

        lax.fori_loop(0, CHUNK, row_body, 0)

    gh = [None] * N_CHUNKS
    oh = [None] * N_CHUNKS
    gh[0] = start_gather(0)
    for g in range(N_CHUNKS):
        gh[g][0].wait()
        gh[g][1].wait()
        if g + 1 < N_CHUNKS:
            gh[g + 1] = start_gather(g + 1)
        if g >= 2:
            oh[g - 2].wait()
        convert_chunk(rows_bufs[g % 2], out_bufs[g % 2])
        oh[g] = pltpu.async_copy(
            out_bufs[g % 2],
            out_hbm.at[pl.ds((wid * N_CHUNKS + g) * 16, 16)], osem)
    oh[N_CHUNKS - 2].wait()
    oh[N_CHUNKS - 1].wait()


def kernel(xBT, embedding):
    idx = xBT.reshape(NW, N_CHUNKS, CHUNK)
    table = embedding.reshape(800000, 128)
    packed = _embed_sc(idx, table)
    out = lax.bitcast_convert_type(packed, jnp.bfloat16)
    return out.reshape(4, 2048, D)


# speedup vs baseline: 1.0745x; 1.0745x over previous
import functools

import jax
import jax.numpy as jnp
from jax import lax
from jax.experimental import pallas as pl
from jax.experimental.pallas import tpu as pltpu, tpu_sc as plsc

D = 1024
DW = D // 2

_info = plsc.get_sparse_core_info()
NC, NS, L = _info.num_cores, _info.num_subcores, _info.num_lanes
NW = NC * NS

B = 4 * 2048
B_PER_W = B // NW
CHUNK = 32
N_CHUNKS = B_PER_W // CHUNK

_mesh = plsc.VectorSubcoreMesh(core_axis_name="c", subcore_axis_name="s")


@functools.partial(
    pl.kernel,
    mesh=_mesh,
    out_type=jax.ShapeDtypeStruct((B * DW // 1024, 8, 128), jnp.int32),
    scratch_types=[
        pltpu.VMEM((2, 128), jnp.int32),
        pltpu.VMEM((CHUNK, D), jnp.float32),
        pltpu.VMEM((CHUNK, D), jnp.float32),
        pltpu.VMEM((16, 8, 128), jnp.int32),
        pltpu.VMEM((16, 8, 128), jnp.int32),
        pltpu.SemaphoreType.DMA,
        pltpu.SemaphoreType.DMA,
    ],
    compiler_params=pltpu.CompilerParams(
        use_tc_tiling_on_sc=True, needs_layout_passes=False),
)
def _embed_sc(idx_hbm, table_hbm, out_hbm, idx_v, rows0, rows1,
              outb0, outb1, gsem, osem):
    wid = lax.axis_index("s") * NC + lax.axis_index("c")
    pltpu.sync_copy(idx_hbm.at[pl.ds(2 * wid, 2)], idx_v)

    iota = lax.iota(jnp.int32, L)
    ev_lane = iota * 2

    rows_bufs = (rows0, rows1)
    out_bufs = (outb0, outb1)

    def start_gather(g):
        ids = idx_v.at[g // 4, pl.ds(32 * (g % 4), 32)]
        return pltpu.async_copy(table_hbm.at[ids], rows_bufs[g % 2], gsem)

    def convert_chunk(rows_ref, out_ref):
        def row_body(r, _):
            rvec = lax.broadcast(r, (L,))
            a_out = r >> 1
            b_par = (r & 1) * 4
            for j in range(32):
                a = plsc.load_gather(rows_ref, [rvec, j * 32 + ev_lane])
                b = plsc.load_gather(rows_ref, [rvec, j * 32 + ev_lane + 1])
                ua = plsc.bitcast(a, jnp.int32)
                ub = plsc.bitcast(b, jnp.int32)
                ta = ua + 0x7FFF + ((ua >> 16) & 1)
                tb = ub + 0x7FFF + ((ub >> 16) & 1)
                word = (lax.shift_right_logical(ta, 16)
                        | (tb & jnp.int32(-0x10000)))
                out_ref[a_out, b_par + j // 8, pl.ds(16 * (j % 8), L)] = word
            return 0

        lax.fori_loop(0, CHUNK, row_body, 0)

    gh = [None] * N_CHUNKS
    oh = [None] * N_CHUNKS
    gh[0] = start_gather(0)
    for g in range(N_CHUNKS):
        gh[g].wait()
        if g + 1 < N_CHUNKS:
            gh[g + 1] = start_gather(g + 1)
        if g >= 2:
            oh[g - 2].wait()
        convert_chunk(rows_bufs[g % 2], out_bufs[g % 2])
        oh[g] = pltpu.async_copy(
            out_bufs[g % 2],
            out_hbm.at[pl.ds((wid * N_CHUNKS + g) * 16, 16)], osem)
    oh[N_CHUNKS - 2].wait()
    oh[N_CHUNKS - 1].wait()


def kernel(xBT, embedding):
    idx = xBT.reshape(64, 128)
    packed = _embed_sc(idx, embedding)
    out = lax.bitcast_convert_type(packed, jnp.bfloat16)
    return out.reshape(4, 2048, D)
